# baseline (speedup 1.0000x reference)
import functools

import jax
import jax.numpy as jnp
from jax import lax
from jax.experimental import pallas as pl
from jax.experimental.pallas import tpu as pltpu
from jax.experimental.pallas import tpu_sc as plsc

NC = 2
NS = 16
CH = 128
NB = 3


def _leaky(v):
    return jnp.where(v >= 0, v, 0.01 * v)


def _build_edge_agg(n_acc, w, kch):
    mesh = plsc.VectorSubcoreMesh(core_axis_name="c", subcore_axis_name="s")
    rpt = n_acc // NS
    zchunks = [(z * CH, CH) for z in range(rpt // CH)]
    if rpt % CH:
        zchunks.append((rpt - rpt % CH, rpt % CH))

    @functools.partial(
        pl.kernel,
        out_type=jax.ShapeDtypeStruct((NC, n_acc, w), jnp.float32),
        mesh=mesh,
        scratch_types=[
            pltpu.VMEM((kch, CH), jnp.int32),
            pltpu.VMEM((kch, CH), jnp.int32),
            [pltpu.VMEM((CH, w), jnp.float32) for _ in range(NB)],
            pltpu.VMEM_SHARED((n_acc, w), jnp.float32),
            [pltpu.SemaphoreType.DMA for _ in range(NB)],
        ],
        compiler_params=pltpu.CompilerParams(use_tc_tiling_on_sc=False),
    )
    def agg(table_hbm, src_hbm, dst_hbm, out_hbm,
            src_v, dst_v, bufs, acc, sems):
        c = lax.axis_index("c")
        s = lax.axis_index("s")
        plane = table_hbm.at[c]
        pltpu.sync_copy(src_hbm.at[pl.ds(s * kch, kch)], src_v)
        pltpu.sync_copy(dst_hbm.at[pl.ds(s * kch, kch)], dst_v)
        zeros16 = jnp.zeros((16,), jnp.float32)

        def zrow(i, carry):
            for j in range(w // 16):
                bufs[0][i, pl.ds(j * 16, 16)] = zeros16
            return carry

        lax.fori_loop(0, CH, zrow, None)
        for zoff, zlen in zchunks:
            pltpu.sync_copy(bufs[0].at[pl.ds(0, zlen)],
                            acc.at[pl.ds(s * rpt + zoff, zlen)])
        plsc.subcore_barrier()

        def start(j, i):
            pltpu.async_copy(plane.at[src_v.at[j]], bufs[i], sems[i])

        def wait(j, i):
            pltpu.make_async_copy(plane.at[src_v.at[j]], bufs[i], sems[i]).wait()

        def scat(j, i):
            pltpu.sync_copy(bufs[i], acc.at[dst_v.at[j]], add=True)

        for i in range(NB):
            start(i, i)

        def body(k, carry):
            j0 = NB * k
            for i in range(NB):
                wait(j0 + i, i)
                scat(j0 + i, i)
                start(j0 + i + NB, i)
            return carry

        lax.fori_loop(0, kch // NB - 1, body, None)
        jlast = kch - NB
        for i in range(NB):
            wait(jlast + i, i)
            scat(jlast + i, i)
        plsc.subcore_barrier()
        pltpu.sync_copy(acc.at[pl.ds(s * rpt, rpt)],
                        out_hbm.at[c, pl.ds(s * rpt, rpt)])

    return agg


def _dense_mid(parts_ref, x_ref, wl_ref, bl_ref, wr_ref, wg_ref, out_ref):
    d = x_ref.shape[1]
    hd = d // 2
    p = parts_ref[...]
    summed = jnp.concatenate([p[0, :, :hd], p[1, :, :hd]], axis=1)
    deg = p[0, :, hd:hd + 1]
    mean = summed / jnp.maximum(deg, 1.0)
    h = (jnp.dot(mean, wl_ref[...], preferred_element_type=jnp.float32)
         + bl_ref[...]
         + jnp.dot(x_ref[...], wr_ref[...], preferred_element_type=jnp.float32))
    h = _leaky(h)
    hw = jnp.dot(h, wg_ref[...], preferred_element_type=jnp.float32)
    out_ref[0] = hw[:, :hd]
    out_ref[1] = hw[:, hd:]


def _final(parts_ref, x_ref, bg_ref, out_ref):
    p = parts_ref[...]
    hw_sum = jnp.concatenate([p[0], p[1]], axis=1)
    out_ref[...] = _leaky(hw_sum + bg_ref[...]) + x_ref[...]


def kernel(x, edge_index, W_sage_l, b_sage_l, W_sage_r, W_gcn, b_gcn):
    n, d = x.shape
    e = edge_index.shape[1]
    hd = d // 2
    w1 = hd + 16

    ept = -(-e // (NS * NB * CH)) * NB * CH
    e_pad = ept * NS
    kch = ept // CH
    n_acc = -(-(n + 1) // NS) * NS

    src = edge_index[0]
    dst = edge_index[1]
    pad = e_pad - e
    src2d = jnp.concatenate([src, jnp.zeros((pad,), jnp.int32)]).reshape(-1, CH)
    dst2d = jnp.concatenate([dst, jnp.full((pad,), n, jnp.int32)]).reshape(-1, CH)
    ones = jnp.ones((n, 16), jnp.float32)
    table1 = jnp.stack(
        [jnp.concatenate([x[:, :hd], ones], axis=1),
         jnp.concatenate([x[:, hd:], ones], axis=1)], axis=0)

    parts1 = _build_edge_agg(n_acc, w1, kch)(table1, src2d, dst2d)

    bn = 1000
    assert n % bn == 0
    hw_split = pl.pallas_call(
        _dense_mid,
        grid=(n // bn,),
        in_specs=[
            pl.BlockSpec((NC, bn, w1), lambda i: (0, i, 0)),
            pl.BlockSpec((bn, d), lambda i: (i, 0)),
            pl.BlockSpec((d, d), lambda i: (0, 0)),
            pl.BlockSpec((1, d), lambda i: (0, 0)),
            pl.BlockSpec((d, d), lambda i: (0, 0)),
            pl.BlockSpec((d, d), lambda i: (0, 0)),
        ],
        out_specs=pl.BlockSpec((NC, bn, hd), lambda i: (0, i, 0)),
        out_shape=jax.ShapeDtypeStruct((NC, n, hd), jnp.float32),
    )(parts1, x, W_sage_l, b_sage_l.reshape(1, d), W_sage_r, W_gcn)

    parts2 = _build_edge_agg(n_acc, hd, kch)(hw_split, src2d, dst2d)

    out = pl.pallas_call(
        _final,
        grid=(n // bn,),
        in_specs=[
            pl.BlockSpec((NC, bn, hd), lambda i: (0, i, 0)),
            pl.BlockSpec((bn, d), lambda i: (i, 0)),
            pl.BlockSpec((1, d), lambda i: (0, 0)),
        ],
        out_specs=pl.BlockSpec((bn, d), lambda i: (i, 0)),
        out_shape=jax.ShapeDtypeStruct((n, d), jnp.float32),
    )(parts2, x, b_gcn.reshape(1, d))
    return out


# speedup vs baseline: 1.3419x; 1.3419x over previous
import functools

import jax
import jax.numpy as jnp
from jax import lax
from jax.experimental import pallas as pl
from jax.experimental.pallas import tpu as pltpu
from jax.experimental.pallas import tpu_sc as plsc

NC = 2
NS = 16
CH = 128
NB = 2


def _leaky(v):
    return jnp.where(v >= 0, v, 0.01 * v)


def _build_edge_agg(n_acc, w, kch, count_deg):
    mesh = plsc.VectorSubcoreMesh(core_axis_name="c", subcore_axis_name="s")
    rpt = n_acc // NS
    zchunks = [(z * CH, CH) for z in range(rpt // CH)]
    if rpt % CH:
        zchunks.append((rpt - rpt % CH, rpt % CH))

    out_types = [jax.ShapeDtypeStruct((NC, n_acc, w), jnp.float32)]
    scratch = [
        pltpu.VMEM((kch, CH), jnp.int32),
        pltpu.VMEM((kch, CH), jnp.int32),
        [pltpu.VMEM((CH, w), jnp.float32) for _ in range(NB)],
        pltpu.VMEM_SHARED((n_acc, w), jnp.float32),
        [pltpu.SemaphoreType.DMA for _ in range(NB)],
    ]
    if count_deg:
        out_types.append(jax.ShapeDtypeStruct((NC, n_acc, 16), jnp.float32))
        scratch += [
            pltpu.VMEM((CH, 16), jnp.float32),
            pltpu.VMEM_SHARED((n_acc, 16), jnp.float32),
        ]

    @functools.partial(
        pl.kernel,
        out_type=tuple(out_types) if count_deg else out_types[0],
        mesh=mesh,
        scratch_types=scratch,
        compiler_params=pltpu.CompilerParams(use_tc_tiling_on_sc=False),
    )
    def agg(table_hbm, src_hbm, dst_hbm, out_hbm, *rest):
        if count_deg:
            deg_hbm, src_v, dst_v, bufs, acc, sems, ones_b, dacc = rest
        else:
            src_v, dst_v, bufs, acc, sems = rest
        c = lax.axis_index("c")
        s = lax.axis_index("s")
        plane = table_hbm.at[c]
        pltpu.sync_copy(src_hbm.at[pl.ds(s * kch, kch)], src_v)
        pltpu.sync_copy(dst_hbm.at[pl.ds(s * kch, kch)], dst_v)
        zeros16 = jnp.zeros((16,), jnp.float32)

        def zrow(i, carry):
            for j in range(w // 16):
                bufs[0][i, pl.ds(j * 16, 16)] = zeros16
            return carry

        lax.fori_loop(0, CH, zrow, None)
        for zoff, zlen in zchunks:
            pltpu.sync_copy(bufs[0].at[pl.ds(0, zlen)],
                            acc.at[pl.ds(s * rpt + zoff, zlen)])
        if count_deg:
            def fill(val):
                def go(i, carry):
                    ones_b[i, pl.ds(0, 16)] = val
                    return carry
                lax.fori_loop(0, CH, go, None)

            fill(zeros16)
            for zoff, zlen in zchunks:
                pltpu.sync_copy(ones_b.at[pl.ds(0, zlen)],
                                dacc.at[pl.ds(s * rpt + zoff, zlen)])
            fill(zeros16 + 1.0)
        plsc.subcore_barrier()

        def start(j, i):
            pltpu.async_copy(plane.at[src_v.at[j]], bufs[i], sems[i])

        def wait(j, i):
            pltpu.make_async_copy(plane.at[src_v.at[j]], bufs[i], sems[i]).wait()

        def scat(j, i):
            pltpu.sync_copy(bufs[i], acc.at[dst_v.at[j]], add=True)

        def deg_scat(j):
            pltpu.sync_copy(ones_b, dacc.at[dst_v.at[j]], add=True)

        for i in range(NB):
            start(i, i)

        def body(k, carry):
            j0 = NB * k
            for i in range(NB):
                wait(j0 + i, i)
                scat(j0 + i, i)
                start(j0 + i + NB, i)
            if count_deg:
                deg_scat(j0 + c)
            return carry

        lax.fori_loop(0, kch // NB - 1, body, None)
        jlast = kch - NB
        for i in range(NB):
            wait(jlast + i, i)
            scat(jlast + i, i)
        if count_deg:
            deg_scat(jlast + c)
        plsc.subcore_barrier()
        pltpu.sync_copy(acc.at[pl.ds(s * rpt, rpt)],
                        out_hbm.at[c, pl.ds(s * rpt, rpt)])
        if count_deg:
            pltpu.sync_copy(dacc.at[pl.ds(s * rpt, rpt)],
                            deg_hbm.at[c, pl.ds(s * rpt, rpt)])

    return agg


def _dense_mid(parts_ref, degs_ref, x_ref, wl_ref, bl_ref, wr_ref, wg_ref,
               out_ref):
    d = x_ref.shape[1]
    hd = d // 2
    p = parts_ref[...]
    summed = jnp.concatenate([p[0], p[1]], axis=1)
    dg = degs_ref[...]
    deg = dg[0, :, 0:1] + dg[1, :, 0:1]
    mean = summed / jnp.maximum(deg, 1.0)
    h = (jnp.dot(mean, wl_ref[...], preferred_element_type=jnp.float32)
         + bl_ref[...]
         + jnp.dot(x_ref[...], wr_ref[...], preferred_element_type=jnp.float32))
    h = _leaky(h)
    hw = jnp.dot(h, wg_ref[...], preferred_element_type=jnp.float32)
    out_ref[0] = hw[:, :hd]
    out_ref[1] = hw[:, hd:]


def _final(parts_ref, x_ref, bg_ref, out_ref):
    p = parts_ref[...]
    hw_sum = jnp.concatenate([p[0], p[1]], axis=1)
    out_ref[...] = _leaky(hw_sum + bg_ref[...]) + x_ref[...]


def kernel(x, edge_index, W_sage_l, b_sage_l, W_sage_r, W_gcn, b_gcn):
    n, d = x.shape
    e = edge_index.shape[1]
    hd = d // 2

    ept = -(-e // (NS * NB * CH)) * NB * CH
    e_pad = ept * NS
    kch = ept // CH
    n_acc = -(-(n + 1) // NS) * NS

    src = edge_index[0]
    dst = edge_index[1]
    pad = e_pad - e
    src2d = jnp.concatenate([src, jnp.zeros((pad,), jnp.int32)]).reshape(-1, CH)
    dst2d = jnp.concatenate([dst, jnp.full((pad,), n, jnp.int32)]).reshape(-1, CH)
    table1 = jnp.stack([x[:, :hd], x[:, hd:]], axis=0)

    parts1, degs = _build_edge_agg(n_acc, hd, kch, True)(table1, src2d, dst2d)

    bn = 1024
    gn = -(-n // bn)
    hw_split = pl.pallas_call(
        _dense_mid,
        grid=(gn,),
        in_specs=[
            pl.BlockSpec((NC, bn, hd), lambda i: (0, i, 0)),
            pl.BlockSpec((NC, bn, 16), lambda i: (0, i, 0)),
            pl.BlockSpec((bn, d), lambda i: (i, 0)),
            pl.BlockSpec((d, d), lambda i: (0, 0)),
            pl.BlockSpec((1, d), lambda i: (0, 0)),
            pl.BlockSpec((d, d), lambda i: (0, 0)),
            pl.BlockSpec((d, d), lambda i: (0, 0)),
        ],
        out_specs=pl.BlockSpec((NC, bn, hd), lambda i: (0, i, 0)),
        out_shape=jax.ShapeDtypeStruct((NC, n, hd), jnp.float32),
    )(parts1, degs, x, W_sage_l, b_sage_l.reshape(1, d), W_sage_r, W_gcn)

    parts2 = _build_edge_agg(n_acc, hd, kch, False)(hw_split, src2d, dst2d)

    out = pl.pallas_call(
        _final,
        grid=(gn,),
        in_specs=[
            pl.BlockSpec((NC, bn, hd), lambda i: (0, i, 0)),
            pl.BlockSpec((bn, d), lambda i: (i, 0)),
            pl.BlockSpec((1, d), lambda i: (0, 0)),
        ],
        out_specs=pl.BlockSpec((bn, d), lambda i: (i, 0)),
        out_shape=jax.ShapeDtypeStruct((n, d), jnp.float32),
    )(parts2, x, b_gcn.reshape(1, d))
    return out
